# baseline (speedup 1.0000x reference)
import functools

import jax
import jax.numpy as jnp
import numpy as np
from jax import lax
from jax.experimental import pallas as pl
from jax.experimental.pallas import tpu as pltpu
from jax.experimental.pallas import tpu_sc as plsc

N_HEADS = 8
D_K = 16
N_BUCKETS = 32
B = 2
N_NODES = 10000
E = 320000
F = N_HEADS * D_K

NC = 2
NS = 16
SLAB = E // NS
CHB = 4000
NB = SLAB // CHB
FPW = F // NS
CHE = 1280
NE = E // CHE


def _bucket_thresholds() -> np.ndarray:
    ts = [0.0]
    for k in range(1, N_BUCKETS):
        s = np.float32(k) / np.float32(N_BUCKETS)
        x = np.float32(np.float64(s) * np.float64(s))
        while np.float32(np.sqrt(x)) < s:
            x = np.nextafter(x, np.float32(np.inf))
        while True:
            y = np.nextafter(x, np.float32(-np.inf))
            if y >= 0 and np.float32(np.sqrt(y)) >= s:
                x = y
            else:
                break
        ts.append(float(x))
    return np.array(ts, dtype=np.float32)


_THRESHOLDS = _bucket_thresholds()


def _sc_body(x_hbm, y_hbm, src_hbm, dst_hbm, embt_hbm, thr_hbm, out_hbm,
             x_v, y_v, src_v, dst_v, thr_v, bk_v, embt_v,
             bkc0, bkc1, bkc2, bkc3, oc0, oc1, oc2, oc3, bk_sh,
             sem0, sem1, sem2, sem3, bsem0, bsem1, bsem2, bsem3):
    c = lax.axis_index("c")
    s = lax.axis_index("s")

    pltpu.sync_copy(x_hbm.at[pl.ds(c * N_NODES, N_NODES)], x_v)
    pltpu.sync_copy(y_hbm.at[pl.ds(c * N_NODES, N_NODES)], y_v)
    pltpu.sync_copy(thr_hbm, thr_v)
    slab = pl.multiple_of(c * E + s * SLAB, 8)

    iota = lax.broadcasted_iota(jnp.int32, (16,), 0)

    for b in range(NB):
        base = pl.multiple_of(slab + b * CHB, 8)
        pltpu.sync_copy(src_hbm.at[pl.ds(base, CHB)], src_v)
        pltpu.sync_copy(dst_hbm.at[pl.ds(base, CHB)], dst_v)

        @plsc.parallel_loop(0, CHB // 16, unroll=4)
        def grp(g, b=b):
            off = g * 16
            sn = src_v[pl.ds(off, 16)]
            dn = dst_v[pl.ds(off, 16)]
            sx = plsc.load_gather(x_v, [sn])
            sy = plsc.load_gather(y_v, [sn])
            tx = plsc.load_gather(x_v, [dn])
            ty = plsc.load_gather(y_v, [dn])
            dx = sx - tx
            dy = sy - ty
            d2 = dx * dx + dy * dy
            k = jnp.zeros((16,), jnp.int32)
            for step in (16, 8, 4, 2, 1):
                cand = k + step
                t = plsc.load_gather(thr_v, [cand])
                k = jnp.where(d2 >= t, cand, k)
            bk_v[pl.ds(b * CHB + off, 16)] = k

    pltpu.sync_copy(bk_v, bk_sh.at[pl.ds(s * SLAB, SLAB)])
    plsc.subcore_barrier()

    pltpu.sync_copy(embt_hbm, embt_v)
    fbase = s * FPW
    rbase = pl.multiple_of(c * F + fbase, 8)

    def do_chunk(bkc, oc, sem, e0):

        @plsc.parallel_loop(0, CHE // 16, unroll=6)
        def grp(g):
            off = g * 16
            bvec = bkc[pl.ds(off, 16)] + fbase * 32
            for f in range(FPW):
                vals = plsc.load_gather(embt_v, [bvec + f * 32])
                oc[f, pl.ds(off, 16)] = vals
        pltpu.async_copy(oc, out_hbm.at[pl.ds(rbase, FPW), pl.ds(e0, CHE)],
                         sem)

    def drain(oc, sem):
        pltpu.make_async_copy(oc, out_hbm.at[pl.ds(0, FPW), pl.ds(0, CHE)],
                              sem).wait()

    bufs = ((bkc0, oc0, sem0, bsem0), (bkc1, oc1, sem1, bsem1),
            (bkc2, oc2, sem2, bsem2), (bkc3, oc3, sem3, bsem3))
    nring = len(bufs)

    def fetch(bkc, bsem, e0):
        pltpu.async_copy(bk_sh.at[pl.ds(e0, CHE)], bkc, bsem)

    def wait_fetch(bkc, bsem):
        pltpu.make_async_copy(bk_sh.at[pl.ds(0, CHE)], bkc, bsem).wait()

    def super_body(i, _):
        e0 = pl.multiple_of(nring * i * CHE, 128)
        for j, (bkc, _, _, bsem) in enumerate(bufs):
            fetch(bkc, bsem, e0 + j * CHE)
        for j, (bkc, oc, sem, bsem) in enumerate(bufs):
            @pl.when(i > 0)
            def _(oc=oc, sem=sem):
                drain(oc, sem)

            wait_fetch(bkc, bsem)
            do_chunk(bkc, oc, sem, e0 + j * CHE)
        return ()

    lax.fori_loop(0, NE // nring, super_body, (), unroll=False)
    for bkc, oc, sem, _ in bufs:
        drain(oc, sem)
    for t in range(NE % nring):
        bkc, oc, sem, bsem = bufs[t]
        fetch(bkc, bsem, (NE - NE % nring + t) * CHE)
        wait_fetch(bkc, bsem)
        do_chunk(bkc, oc, sem, (NE - NE % nring + t) * CHE)
    for t in range(NE % nring):
        _, oc, sem, _ = bufs[t]
        drain(oc, sem)


@jax.jit
def _rpe_sc(x1d, y1d, src1d, dst1d, embt1d, thr):
    mesh = plsc.VectorSubcoreMesh(core_axis_name="c", subcore_axis_name="s")
    f = pl.kernel(
        _sc_body,
        out_type=jax.ShapeDtypeStruct((B * F, E), jnp.float32),
        mesh=mesh,
        compiler_params=pltpu.CompilerParams(needs_layout_passes=False),
        scratch_types=[
            pltpu.VMEM((N_NODES,), jnp.float32),
            pltpu.VMEM((N_NODES,), jnp.float32),
            pltpu.VMEM((CHB,), jnp.int32),
            pltpu.VMEM((CHB,), jnp.int32),
            pltpu.VMEM((N_BUCKETS,), jnp.float32),
            pltpu.VMEM((SLAB,), jnp.int32),
            pltpu.VMEM((N_BUCKETS * F,), jnp.float32),
            pltpu.VMEM((CHE,), jnp.int32),
            pltpu.VMEM((CHE,), jnp.int32),
            pltpu.VMEM((CHE,), jnp.int32),
            pltpu.VMEM((CHE,), jnp.int32),
            pltpu.VMEM((FPW, CHE), jnp.float32),
            pltpu.VMEM((FPW, CHE), jnp.float32),
            pltpu.VMEM((FPW, CHE), jnp.float32),
            pltpu.VMEM((FPW, CHE), jnp.float32),
            pltpu.VMEM_SHARED((E,), jnp.int32),
            pltpu.SemaphoreType.DMA,
            pltpu.SemaphoreType.DMA,
            pltpu.SemaphoreType.DMA,
            pltpu.SemaphoreType.DMA,
            pltpu.SemaphoreType.DMA,
            pltpu.SemaphoreType.DMA,
            pltpu.SemaphoreType.DMA,
            pltpu.SemaphoreType.DMA,
        ],
    )
    return f(x1d, y1d, src1d, dst1d, embt1d, thr)


def kernel(coords, edge_index, embedding):
    coords = coords.astype(jnp.float32)
    ei = edge_index.astype(jnp.int32)
    x1d = coords[:, :, 0].reshape(B * N_NODES)
    y1d = coords[:, :, 1].reshape(B * N_NODES)
    src1d = ei[:, :, 0].reshape(B * E)
    dst1d = ei[:, :, 1].reshape(B * E)
    embt1d = embedding.astype(jnp.float32).T.reshape(N_BUCKETS * F)
    thr = jnp.asarray(_THRESHOLDS)
    out = _rpe_sc(x1d, y1d, src1d, dst1d, embt1d, thr)
    return out.reshape(B, N_HEADS, D_K, E).transpose(0, 3, 1, 2)


# speedup vs baseline: 1.0273x; 1.0273x over previous
import functools

import jax
import jax.numpy as jnp
import numpy as np
from jax import lax
from jax.experimental import pallas as pl
from jax.experimental.pallas import tpu as pltpu
from jax.experimental.pallas import tpu_sc as plsc

N_HEADS = 8
D_K = 16
N_BUCKETS = 32
B = 2
N_NODES = 10000
E = 320000
F = N_HEADS * D_K

NC = 2
NS = 16
SLAB = E // NS
CHB = 4000
NB = SLAB // CHB
FPW = F // NS
CHE = 1280
NE = E // CHE


def _bucket_thresholds() -> np.ndarray:
    ts = [0.0]
    for k in range(1, N_BUCKETS):
        s = np.float32(k) / np.float32(N_BUCKETS)
        x = np.float32(np.float64(s) * np.float64(s))
        while np.float32(np.sqrt(x)) < s:
            x = np.nextafter(x, np.float32(np.inf))
        while True:
            y = np.nextafter(x, np.float32(-np.inf))
            if y >= 0 and np.float32(np.sqrt(y)) >= s:
                x = y
            else:
                break
        ts.append(float(x))
    return np.array(ts, dtype=np.float32)


_THRESHOLDS = _bucket_thresholds()


def _sc_body(x_hbm, y_hbm, src_hbm, dst_hbm, embt_hbm, thr_hbm, out_hbm,
             x_v, y_v, src_v, dst_v, thr_v, bk_v, embt_v,
             bkc0, bkc1, bkc2, bkc3, oc0, oc1, oc2, oc3, bk_sh,
             sem0, sem1, sem2, sem3, bsem0, bsem1, bsem2, bsem3):
    c = lax.axis_index("c")
    s = lax.axis_index("s")

    pltpu.sync_copy(x_hbm.at[pl.ds(c * N_NODES, N_NODES)], x_v)
    pltpu.sync_copy(y_hbm.at[pl.ds(c * N_NODES, N_NODES)], y_v)
    pltpu.sync_copy(thr_hbm, thr_v)
    slab = pl.multiple_of(c * E + s * SLAB, 8)

    iota = lax.broadcasted_iota(jnp.int32, (16,), 0)

    for b in range(NB):
        base = pl.multiple_of(slab + b * CHB, 8)
        pltpu.sync_copy(src_hbm.at[pl.ds(base, CHB)], src_v)
        pltpu.sync_copy(dst_hbm.at[pl.ds(base, CHB)], dst_v)

        @plsc.parallel_loop(0, CHB // 16, unroll=4)
        def grp(g, b=b):
            off = g * 16
            sn = src_v[pl.ds(off, 16)]
            dn = dst_v[pl.ds(off, 16)]
            sx = plsc.load_gather(x_v, [sn])
            sy = plsc.load_gather(y_v, [sn])
            tx = plsc.load_gather(x_v, [dn])
            ty = plsc.load_gather(y_v, [dn])
            dx = sx - tx
            dy = sy - ty
            d2 = dx * dx + dy * dy
            k = jnp.zeros((16,), jnp.int32)
            for step in (16, 8, 4, 2, 1):
                cand = k + step
                t = plsc.load_gather(thr_v, [cand])
                k = jnp.where(d2 >= t, cand, k)
            bk_v[pl.ds(b * CHB + off, 16)] = k

    pltpu.sync_copy(bk_v, bk_sh.at[pl.ds(s * SLAB, SLAB)])
    plsc.subcore_barrier()

    pltpu.sync_copy(embt_hbm, embt_v)
    fbase = s * FPW
    rbase = pl.multiple_of(c * F + fbase, 8)

    def do_chunk(bkc, oc, sem, e0):

        @plsc.parallel_loop(0, CHE // 16, unroll=4)
        def grp(g):
            off = g * 16
            bvec = bkc[pl.ds(off, 16)] + fbase * 32
            for f in range(FPW):
                vals = plsc.load_gather(embt_v, [bvec + f * 32])
                oc[f, pl.ds(off, 16)] = vals
        pltpu.async_copy(oc, out_hbm.at[pl.ds(rbase, FPW), pl.ds(e0, CHE)],
                         sem)

    def drain(oc, sem):
        pltpu.make_async_copy(oc, out_hbm.at[pl.ds(0, FPW), pl.ds(0, CHE)],
                              sem).wait()

    bufs = ((bkc0, oc0, sem0, bsem0), (bkc1, oc1, sem1, bsem1),
            (bkc2, oc2, sem2, bsem2), (bkc3, oc3, sem3, bsem3))
    nring = len(bufs)

    def fetch(bkc, bsem, e0):
        pltpu.async_copy(bk_sh.at[pl.ds(e0, CHE)], bkc, bsem)

    def wait_fetch(bkc, bsem):
        pltpu.make_async_copy(bk_sh.at[pl.ds(0, CHE)], bkc, bsem).wait()

    def super_body(i, _):
        e0 = pl.multiple_of(nring * i * CHE, 128)
        for j, (bkc, _, _, bsem) in enumerate(bufs):
            fetch(bkc, bsem, e0 + j * CHE)
        for j, (bkc, oc, sem, bsem) in enumerate(bufs):
            @pl.when(i > 0)
            def _(oc=oc, sem=sem):
                drain(oc, sem)

            wait_fetch(bkc, bsem)
            do_chunk(bkc, oc, sem, e0 + j * CHE)
        return ()

    lax.fori_loop(0, NE // nring, super_body, (), unroll=False)
    for bkc, oc, sem, _ in bufs:
        drain(oc, sem)
    for t in range(NE % nring):
        bkc, oc, sem, bsem = bufs[t]
        fetch(bkc, bsem, (NE - NE % nring + t) * CHE)
        wait_fetch(bkc, bsem)
        do_chunk(bkc, oc, sem, (NE - NE % nring + t) * CHE)
    for t in range(NE % nring):
        _, oc, sem, _ = bufs[t]
        drain(oc, sem)


@jax.jit
def _rpe_sc(x1d, y1d, src1d, dst1d, embt1d, thr):
    mesh = plsc.VectorSubcoreMesh(core_axis_name="c", subcore_axis_name="s")
    f = pl.kernel(
        _sc_body,
        out_type=jax.ShapeDtypeStruct((B * F, E), jnp.float32),
        mesh=mesh,
        compiler_params=pltpu.CompilerParams(needs_layout_passes=False),
        scratch_types=[
            pltpu.VMEM((N_NODES,), jnp.float32),
            pltpu.VMEM((N_NODES,), jnp.float32),
            pltpu.VMEM((CHB,), jnp.int32),
            pltpu.VMEM((CHB,), jnp.int32),
            pltpu.VMEM((N_BUCKETS,), jnp.float32),
            pltpu.VMEM((SLAB,), jnp.int32),
            pltpu.VMEM((N_BUCKETS * F,), jnp.float32),
            pltpu.VMEM((CHE,), jnp.int32),
            pltpu.VMEM((CHE,), jnp.int32),
            pltpu.VMEM((CHE,), jnp.int32),
            pltpu.VMEM((CHE,), jnp.int32),
            pltpu.VMEM((FPW, CHE), jnp.float32),
            pltpu.VMEM((FPW, CHE), jnp.float32),
            pltpu.VMEM((FPW, CHE), jnp.float32),
            pltpu.VMEM((FPW, CHE), jnp.float32),
            pltpu.VMEM_SHARED((E,), jnp.int32),
            pltpu.SemaphoreType.DMA,
            pltpu.SemaphoreType.DMA,
            pltpu.SemaphoreType.DMA,
            pltpu.SemaphoreType.DMA,
            pltpu.SemaphoreType.DMA,
            pltpu.SemaphoreType.DMA,
            pltpu.SemaphoreType.DMA,
            pltpu.SemaphoreType.DMA,
        ],
    )
    return f(x1d, y1d, src1d, dst1d, embt1d, thr)


def kernel(coords, edge_index, embedding):
    coords = coords.astype(jnp.float32)
    ei = edge_index.astype(jnp.int32)
    x1d = coords[:, :, 0].reshape(B * N_NODES)
    y1d = coords[:, :, 1].reshape(B * N_NODES)
    src1d = ei[:, :, 0].reshape(B * E)
    dst1d = ei[:, :, 1].reshape(B * E)
    embt1d = embedding.astype(jnp.float32).T.reshape(N_BUCKETS * F)
    thr = jnp.asarray(_THRESHOLDS)
    out = _rpe_sc(x1d, y1d, src1d, dst1d, embt1d, thr)
    return out.reshape(B, N_HEADS, D_K, E).transpose(0, 3, 1, 2)
